# baseline (speedup 1.0000x reference)
import functools

import jax
import jax.numpy as jnp
from jax import lax
from jax.experimental import pallas as pl
from jax.experimental.pallas import tpu as pltpu

NUM_NODES = 1000000
EMBED_SIZE = 64
BATCH = 16384

_NBLK = 8
_BLOCK = BATCH // _NBLK
_DEPTH = 16


def _gather_kernel(idx_ref, table_ref, out_ref, sems):
    i = pl.program_id(0)
    base = i * _BLOCK

    def row_copy(j, n):
        return pltpu.make_async_copy(
            table_ref.at[pl.ds(n, 1), :],
            out_ref.at[pl.ds(j, 1), :],
            sems.at[j % _DEPTH],
        )

    def body(j, _):
        row_copy(j, idx_ref[base + j]).start()

        @pl.when(j >= _DEPTH)
        def _wait():
            jd = j - _DEPTH
            row_copy(jd, idx_ref[base + jd]).wait()

        return _

    lax.fori_loop(0, _BLOCK, body, 0)
    for d in range(_DEPTH):
        jd = _BLOCK - _DEPTH + d
        row_copy(jd, idx_ref[base + jd]).wait()


@jax.jit
def _gather(node_embed_weight, node_ids):
    grid_spec = pltpu.PrefetchScalarGridSpec(
        num_scalar_prefetch=1,
        grid=(_NBLK,),
        in_specs=[pl.BlockSpec(memory_space=pl.ANY)],
        out_specs=pl.BlockSpec((_BLOCK, EMBED_SIZE), lambda i, idx_ref: (i, 0)),
        scratch_shapes=[pltpu.SemaphoreType.DMA((_DEPTH,))],
    )
    return pl.pallas_call(
        _gather_kernel,
        grid_spec=grid_spec,
        out_shape=jax.ShapeDtypeStruct((BATCH, EMBED_SIZE), jnp.float32),
        compiler_params=pltpu.CompilerParams(
            dimension_semantics=("arbitrary",),
        ),
    )(node_ids, node_embed_weight)


def kernel(node_ids, node_tids, type_ids, node_embed_weight):
    del node_tids, type_ids
    return _gather(node_embed_weight, node_ids)


# speedup vs baseline: 1.8458x; 1.8458x over previous
import functools

import jax
import jax.numpy as jnp
from jax import lax
from jax.experimental import pallas as pl
from jax.experimental.pallas import tpu as pltpu

NUM_NODES = 1000000
EMBED_SIZE = 64
BATCH = 16384

_NBLK = 8
_BLOCK = BATCH // _NBLK
_DEPTH = 64


def _gather_kernel(idx_ref, table_ref, out_ref, sems):
    i = pl.program_id(0)
    base = i * _BLOCK

    def fire(j):
        pltpu.make_async_copy(
            table_ref.at[pl.ds(idx_ref[base + j], 1), :],
            out_ref.at[pl.ds(j, 1), :],
            sems.at[lax.bitwise_and(j, _DEPTH - 1)],
        ).start()

    def drain(slot):
        pltpu.make_async_copy(
            table_ref.at[pl.ds(0, 1), :],
            out_ref.at[pl.ds(0, 1), :],
            sems.at[slot],
        ).wait()

    def prologue(j, _):
        fire(j)
        return _

    lax.fori_loop(0, _DEPTH, prologue, 0, unroll=8)

    def steady(j, _):
        drain(lax.bitwise_and(j, _DEPTH - 1))
        fire(j)
        return _

    lax.fori_loop(_DEPTH, _BLOCK, steady, 0, unroll=8)

    def epilogue(s, _):
        drain(s)
        return _

    lax.fori_loop(0, _DEPTH, epilogue, 0, unroll=8)


@jax.jit
def _gather(node_embed_weight, node_ids):
    grid_spec = pltpu.PrefetchScalarGridSpec(
        num_scalar_prefetch=1,
        grid=(_NBLK,),
        in_specs=[pl.BlockSpec(memory_space=pl.ANY)],
        out_specs=pl.BlockSpec((_BLOCK, EMBED_SIZE), lambda i, idx_ref: (i, 0)),
        scratch_shapes=[pltpu.SemaphoreType.DMA((_DEPTH,))],
    )
    return pl.pallas_call(
        _gather_kernel,
        grid_spec=grid_spec,
        out_shape=jax.ShapeDtypeStruct((BATCH, EMBED_SIZE), jnp.float32),
        compiler_params=pltpu.CompilerParams(
            dimension_semantics=("arbitrary",),
        ),
    )(node_ids, node_embed_weight)


def kernel(node_ids, node_tids, type_ids, node_embed_weight):
    del node_tids, type_ids
    return _gather(node_embed_weight, node_ids)


# speedup vs baseline: 2.1389x; 1.1588x over previous
import functools

import jax
import jax.numpy as jnp
from jax import lax
from jax.experimental import pallas as pl
from jax.experimental.pallas import tpu as pltpu

NUM_NODES = 1000000
EMBED_SIZE = 64
BATCH = 16384

_NBLK = 8
_BLOCK = BATCH // _NBLK
_W = 32
_NS = 4
_NWIN = _BLOCK // _W


def _gather_kernel(idx_ref, table_ref, out_ref, sems):
    i = pl.program_id(0)
    base = i * _BLOCK

    def fire_window(w):
        sem = sems.at[lax.bitwise_and(w, _NS - 1)]

        def fire(k, _):
            j = w * _W + k
            pltpu.make_async_copy(
                table_ref.at[pl.ds(idx_ref[base + j], 1), :],
                out_ref.at[pl.ds(j, 1), :],
                sem,
            ).start()
            return _

        lax.fori_loop(0, _W, fire, 0, unroll=True)

    def drain_window(w):
        pltpu.make_async_copy(
            table_ref.at[pl.ds(0, _W), :],
            out_ref.at[pl.ds(0, _W), :],
            sems.at[lax.bitwise_and(w, _NS - 1)],
        ).wait()

    def prologue(w, _):
        fire_window(w)
        return _

    lax.fori_loop(0, _NS, prologue, 0)

    def steady(w, _):
        drain_window(w)
        fire_window(w)
        return _

    lax.fori_loop(_NS, _NWIN, steady, 0)

    def epilogue(w, _):
        drain_window(w)
        return _

    lax.fori_loop(_NWIN - _NS, _NWIN, epilogue, 0)


@jax.jit
def _gather(node_embed_weight, node_ids):
    grid_spec = pltpu.PrefetchScalarGridSpec(
        num_scalar_prefetch=1,
        grid=(_NBLK,),
        in_specs=[pl.BlockSpec(memory_space=pl.ANY)],
        out_specs=pl.BlockSpec((_BLOCK, EMBED_SIZE), lambda i, idx_ref: (i, 0)),
        scratch_shapes=[pltpu.SemaphoreType.DMA((_NS,))],
    )
    return pl.pallas_call(
        _gather_kernel,
        grid_spec=grid_spec,
        out_shape=jax.ShapeDtypeStruct((BATCH, EMBED_SIZE), jnp.float32),
        compiler_params=pltpu.CompilerParams(
            dimension_semantics=("arbitrary",),
        ),
    )(node_ids, node_embed_weight)


def kernel(node_ids, node_tids, type_ids, node_embed_weight):
    del node_tids, type_ids
    return _gather(node_embed_weight, node_ids)
